# baseline (speedup 1.0000x reference)
import functools

import jax
import jax.numpy as jnp
from jax import lax
from jax.experimental import pallas as pl
from jax.experimental.pallas import tpu as pltpu
from jax.experimental.pallas import tpu_sc as plsc

D = 128
S = 200
B = 4096
CH = 128
NC, NS = 2, 16
NW = NC * NS
NROWS = B * S
NCHUNK = NROWS // CH
CPW = NCHUNK // NW

_mesh = plsc.VectorSubcoreMesh(core_axis_name="c", subcore_axis_name="s")


@functools.partial(
    pl.kernel,
    mesh=_mesh,
    compiler_params=pltpu.CompilerParams(use_tc_tiling_on_sc=False, needs_layout_passes=False),
    out_type=jax.ShapeDtypeStruct((NROWS, D), jnp.float32),
    scratch_types=[
        pltpu.VMEM((CPW, CH), jnp.int32),
        pltpu.VMEM((CPW, CH), jnp.int32),
        pltpu.VMEM((S, D), jnp.float32),
        pltpu.VMEM((CH, D // 2), jnp.int32),
        pltpu.VMEM((CH, D // 2), jnp.int32),
        pltpu.VMEM((CH, D), jnp.float32),
        pltpu.VMEM((CH, D), jnp.float32),
        pltpu.SemaphoreType.DMA,
        pltpu.SemaphoreType.DMA,
        pltpu.SemaphoreType.DMA,
    ],
)
def _sc_embed(ids_hbm, oidx_hbm, table_hbm, pos_hbm, out_hbm,
              idx_v, oidx_v, pos_v, g0, g1, f0, f1, gsem, o0, o1):
    gbuf = (g0, g1)
    fbuf = (f0, f1)
    osem = (o0, o1)
    wid = lax.axis_index("s") * NC + lax.axis_index("c")
    chunk0 = wid * CPW

    def start_gather(c_local, b):
        pltpu.async_copy(table_hbm.at[idx_v.at[c_local]], gbuf[b], gsem)

    def wait_gather(b):
        pltpu.make_async_copy(table_hbm.at[pl.ds(0, CH)], gbuf[b], gsem).wait()

    def start_scatter(c_local, b):
        pltpu.async_copy(fbuf[b], out_hbm.at[oidx_v.at[c_local]], osem[b])

    def wait_scatter(b):
        pltpu.make_async_copy(fbuf[b], out_hbm.at[pl.ds(0, CH)], osem[b]).wait()

    pltpu.sync_copy(ids_hbm.at[pl.ds(chunk0, CPW)], idx_v)
    start_gather(0, 0)
    pltpu.sync_copy(oidx_hbm.at[pl.ds(chunk0, CPW)], oidx_v)
    pltpu.sync_copy(pos_hbm.at[pl.ds(0, S)], pos_v)

    def outer(g, carry):
        for b in range(2):
            c = g * 2 + b
            wait_gather(b)

            @pl.when(c + 1 < CPW)
            def _(_c=c, _b=b):
                start_gather(_c + 1, 1 - _b)

            @pl.when(c >= 2)
            def _(_b=b):
                wait_scatter(_b)

            pv = [pos_v[c, pl.ds(k * 16, 16)] for k in range(D // 16)]

            def add_rows(j, carry2, _b=b, _pv=pv):
                for u in range(2):
                    jj = j * 2 + u
                    for k in range(D // 32):
                        iv = gbuf[_b][jj, pl.ds(k * 16, 16)]
                        bv = plsc.bitcast(iv, jnp.bfloat16)
                        lo, hi = plsc.unpack(
                            bv, format=plsc.PackFormat.INTERLEAVED,
                            preferred_element_type=jnp.float32)
                        fbuf[_b][jj, pl.ds(k * 32, 16)] = lo + _pv[2 * k]
                        fbuf[_b][jj, pl.ds(k * 32 + 16, 16)] = hi + _pv[2 * k + 1]
                return carry2

            lax.fori_loop(0, CH // 2, add_rows, 0)
            start_scatter(c, b)
        return carry

    lax.fori_loop(0, CPW // 2, outer, 0)
    for b in range(2):
        wait_scatter(b)


def kernel(token_ids, token_table, pos_table):
    bsz, seq = token_ids.shape
    vocab = token_table.shape[0]
    table_bf = (token_table.astype(jnp.bfloat16)
                .reshape(vocab, D // 32, 2, 16)
                .swapaxes(2, 3)
                .reshape(vocab, D))
    table_pk = lax.bitcast_convert_type(
        table_bf.reshape(vocab, D // 2, 2), jnp.int32)
    ids = (token_ids.astype(jnp.int32).T
           .reshape(S, NW, CH)
           .transpose(1, 0, 2)
           .reshape(NCHUNK, CH))
    bidx = (jnp.arange(NW, dtype=jnp.int32)[:, None, None] * CH
            + jnp.arange(CH, dtype=jnp.int32)[None, None, :])
    oidx = (bidx * S
            + jnp.arange(S, dtype=jnp.int32)[None, :, None]
            ).reshape(NCHUNK, CH)
    out = _sc_embed(ids, oidx, table_pk, pos_table)
    return out.reshape(bsz, seq, token_table.shape[1])
